# baseline (speedup 1.0000x reference)
import functools

import jax
import jax.numpy as jnp
from jax import lax
from jax.experimental import pallas as pl
from jax.experimental.pallas import tpu as pltpu
from jax.experimental.pallas import tpu_sc as plsc

N = 10000
E = 320000
D = 128
NLAYERS = 3

NC = 2
NS = 16
NW = NC * NS
EPW = E // NW
K = 80
ITERS = EPW // K
ROWS_PER_IO = 624
TAIL_BASE = ROWS_PER_IO * NS
TAIL_ROWS = N - TAIL_BASE


def _sc_agg_body(h_hbm, src_hbm, dst_hbm, out_hbm,
                 src_v, dst_v, rows_v, acc_sh,
                 gsem0, gsem1, isem):
    gsem = (gsem0, gsem1)
    cid = lax.axis_index("c")
    sid = lax.axis_index("s")
    wid = sid * NC + cid

    base = wid * EPW
    idx_cps = [pltpu.async_copy(src_hbm.at[pl.ds(base, EPW)], src_v, isem)]
    for i in range(ITERS):
        idx_cps.append(pltpu.async_copy(
            dst_hbm.at[pl.ds(base + i * K, K)], dst_v.at[i], isem))

    zv = jnp.zeros((16,), jnp.float32)

    def zero_row(r, _):
        for c8 in range(D // 16):
            rows_v[0, r, pl.ds(c8 * 16, 16)] = zv
        return 0

    lax.fori_loop(0, K, zero_row, 0)

    for j in range(ROWS_PER_IO // K):
        pltpu.sync_copy(
            rows_v.at[0], acc_sh.at[pl.ds(sid * ROWS_PER_IO + j * K, K)])
    rem = ROWS_PER_IO % K
    if rem:
        pltpu.sync_copy(
            rows_v.at[0, pl.ds(0, rem)],
            acc_sh.at[pl.ds(sid * ROWS_PER_IO + (ROWS_PER_IO // K) * K, rem)])

    @pl.when(sid == NS - 1)
    def _zero_tail():
        pltpu.sync_copy(rows_v.at[0, pl.ds(0, TAIL_ROWS)],
                        acc_sh.at[pl.ds(TAIL_BASE, TAIL_ROWS)])

    def fire_g(i, b):
        pltpu.async_copy(
            h_hbm.at[src_v.at[pl.ds(i * K, K)]], rows_v.at[b], gsem[b])

    def wait_g(i, b):
        pltpu.make_async_copy(
            h_hbm.at[src_v.at[pl.ds(i * K, K)]], rows_v.at[b],
            gsem[b]).wait()

    def scatter(i, b):
        pltpu.sync_copy(rows_v.at[b], acc_sh.at[dst_v.at[i]], add=True)

    for cp in idx_cps:
        cp.wait()
    fire_g(0, 0)
    fire_g(1, 1)
    plsc.subcore_barrier()

    def body(j, _):
        i0 = 2 * j
        wait_g(i0, 0)
        scatter(i0, 0)
        fire_g(i0 + 2, 0)
        wait_g(i0 + 1, 1)
        scatter(i0 + 1, 1)
        fire_g(i0 + 3, 1)
        return 0

    lax.fori_loop(0, (ITERS - 5) // 2, body, 0)
    wait_g(ITERS - 5, 0)
    scatter(ITERS - 5, 0)
    fire_g(ITERS - 3, 0)
    wait_g(ITERS - 4, 1)
    scatter(ITERS - 4, 1)
    fire_g(ITERS - 2, 1)
    wait_g(ITERS - 3, 0)
    scatter(ITERS - 3, 0)
    fire_g(ITERS - 1, 0)
    wait_g(ITERS - 2, 1)
    scatter(ITERS - 2, 1)
    wait_g(ITERS - 1, 0)
    scatter(ITERS - 1, 0)
    plsc.subcore_barrier()

    pltpu.sync_copy(
        acc_sh.at[pl.ds(sid * ROWS_PER_IO, ROWS_PER_IO)],
        out_hbm.at[pl.ds(cid * N + sid * ROWS_PER_IO, ROWS_PER_IO)],
    )

    @pl.when(sid == NS - 1)
    def _copy_tail():
        pltpu.sync_copy(
            acc_sh.at[pl.ds(TAIL_BASE, TAIL_ROWS)],
            out_hbm.at[pl.ds(cid * N + TAIL_BASE, TAIL_ROWS)],
        )


def _sc_agg(h, src, dst):
    mesh = plsc.VectorSubcoreMesh(core_axis_name="c", subcore_axis_name="s")
    return pl.kernel(
        _sc_agg_body,
        out_type=jax.ShapeDtypeStruct((2 * N, D), jnp.float32),
        mesh=mesh,
        scratch_types=[
            pltpu.VMEM((EPW,), jnp.int32),
            pltpu.VMEM((ITERS, K), jnp.int32),
            pltpu.VMEM((2, K, D), jnp.float32),
            pltpu.VMEM_SHARED((N, D), jnp.float32),
            pltpu.SemaphoreType.DMA,
            pltpu.SemaphoreType.DMA,
            pltpu.SemaphoreType.DMA,
        ],
    )(h, src, dst)


BM = 2000


def _enc_body(x_ref, w_ref, b_ref, o_ref):
    o_ref[...] = jnp.maximum(
        jnp.dot(x_ref[...], w_ref[...], preferred_element_type=jnp.float32)
        + b_ref[...], 0.0)


def _tc_encoder(x, W_enc, b_enc):
    return pl.pallas_call(
        _enc_body,
        grid=(N // BM,),
        in_specs=[
            pl.BlockSpec((BM, D), lambda i: (i, 0)),
            pl.BlockSpec((D, D), lambda i: (0, 0)),
            pl.BlockSpec((1, D), lambda i: (0, 0)),
        ],
        out_specs=pl.BlockSpec((BM, D), lambda i: (i, 0)),
        out_shape=jax.ShapeDtypeStruct((N, D), jnp.float32),
    )(x, W_enc, b_enc.reshape(1, D))


def _tc_mlp(h, agg2, W1, b1, W2, b2, eps, l):
    def body(eps_ref, h_ref, a0_ref, a1_ref, w1_ref, b1_ref, w2_ref,
             b2_ref, o_ref):
        scale = 1.0 + eps_ref[l]
        z = scale * h_ref[...] + a0_ref[...] + a1_ref[...]
        z = jnp.maximum(
            jnp.dot(z, w1_ref[0], preferred_element_type=jnp.float32)
            + b1_ref[l], 0.0)
        o_ref[...] = jnp.maximum(
            jnp.dot(z, w2_ref[0], preferred_element_type=jnp.float32)
            + b2_ref[l], 0.0)

    return pl.pallas_call(
        body,
        grid=(N // BM,),
        in_specs=[
            pl.BlockSpec(memory_space=pltpu.SMEM),
            pl.BlockSpec((BM, D), lambda i: (i, 0)),
            pl.BlockSpec((BM, D), lambda i: (i, 0)),
            pl.BlockSpec((BM, D), lambda i: (i + N // BM, 0)),
            pl.BlockSpec((1, D, D), lambda i: (l, 0, 0)),
            pl.BlockSpec((NLAYERS, D), lambda i: (0, 0)),
            pl.BlockSpec((1, D, D), lambda i: (l, 0, 0)),
            pl.BlockSpec((NLAYERS, D), lambda i: (0, 0)),
        ],
        out_specs=pl.BlockSpec((BM, D), lambda i: (i, 0)),
        out_shape=jax.ShapeDtypeStruct((N, D), jnp.float32),
    )(eps, h, agg2, agg2, W1, b1, W2, b2)


def kernel(x, edge_index, W_enc, b_enc, W1, b1, W2, b2, eps):
    src = edge_index[0].astype(jnp.int32)
    dst = edge_index[1].astype(jnp.int32)
    h = _tc_encoder(x, W_enc, b_enc)
    for l in range(NLAYERS):
        agg2 = _sc_agg(h, src, dst)
        h = _tc_mlp(h, agg2, W1, b1, W2, b2, eps, l)
    return h


# speedup vs baseline: 1.0290x; 1.0290x over previous
import functools

import jax
import jax.numpy as jnp
from jax import lax
from jax.experimental import pallas as pl
from jax.experimental.pallas import tpu as pltpu
from jax.experimental.pallas import tpu_sc as plsc

N = 10000
E = 320000
D = 128
NLAYERS = 3

NC = 2
NS = 16
NW = NC * NS
EPW = E // NW
K = 80
ITERS = EPW // K
ROWS_PER_IO = 624
TAIL_BASE = ROWS_PER_IO * NS
TAIL_ROWS = N - TAIL_BASE


def _sc_agg_body(h_hbm, src_hbm, dst_hbm, out_hbm,
                 src_v, dst_v, rows_v, acc_sh,
                 gsem0, gsem1, isem):
    gsem = (gsem0, gsem1)
    cid = lax.axis_index("c")
    sid = lax.axis_index("s")
    wid = sid * NC + cid

    base = wid * EPW
    idx_cps = [pltpu.async_copy(src_hbm.at[pl.ds(base, EPW)], src_v, isem)]
    for i in range(ITERS):
        idx_cps.append(pltpu.async_copy(
            dst_hbm.at[pl.ds(base + i * K, K)], dst_v.at[i], isem))

    zv = jnp.zeros((16,), jnp.float32)

    def zero_row(r, _):
        for c8 in range(D // 16):
            rows_v[0, r, pl.ds(c8 * 16, 16)] = zv
        return 0

    lax.fori_loop(0, K, zero_row, 0)

    for j in range(ROWS_PER_IO // K):
        pltpu.sync_copy(
            rows_v.at[0], acc_sh.at[pl.ds(sid * ROWS_PER_IO + j * K, K)])
    rem = ROWS_PER_IO % K
    if rem:
        pltpu.sync_copy(
            rows_v.at[0, pl.ds(0, rem)],
            acc_sh.at[pl.ds(sid * ROWS_PER_IO + (ROWS_PER_IO // K) * K, rem)])

    @pl.when(sid == NS - 1)
    def _zero_tail():
        pltpu.sync_copy(rows_v.at[0, pl.ds(0, TAIL_ROWS)],
                        acc_sh.at[pl.ds(TAIL_BASE, TAIL_ROWS)])

    def fire_g(i, b):
        pltpu.async_copy(
            h_hbm.at[src_v.at[pl.ds(i * K, K)]], rows_v.at[b], gsem[b])

    def wait_g(i, b):
        pltpu.make_async_copy(
            h_hbm.at[src_v.at[pl.ds(i * K, K)]], rows_v.at[b],
            gsem[b]).wait()

    def scatter(i, b):
        pltpu.sync_copy(rows_v.at[b], acc_sh.at[dst_v.at[i]], add=True)

    for cp in idx_cps:
        cp.wait()
    fire_g(0, 0)
    fire_g(1, 1)
    plsc.subcore_barrier()

    def body(j, _):
        i0 = 2 * j
        wait_g(i0, 0)
        scatter(i0, 0)
        fire_g(i0 + 2, 0)
        wait_g(i0 + 1, 1)
        scatter(i0 + 1, 1)
        fire_g(i0 + 3, 1)
        return 0

    lax.fori_loop(0, (ITERS - 5) // 2, body, 0)
    wait_g(ITERS - 5, 0)
    scatter(ITERS - 5, 0)
    fire_g(ITERS - 3, 0)
    wait_g(ITERS - 4, 1)
    scatter(ITERS - 4, 1)
    fire_g(ITERS - 2, 1)
    wait_g(ITERS - 3, 0)
    scatter(ITERS - 3, 0)
    fire_g(ITERS - 1, 0)
    wait_g(ITERS - 2, 1)
    scatter(ITERS - 2, 1)
    wait_g(ITERS - 1, 0)
    scatter(ITERS - 1, 0)
    plsc.subcore_barrier()

    pltpu.sync_copy(
        acc_sh.at[pl.ds(sid * ROWS_PER_IO, ROWS_PER_IO)],
        out_hbm.at[pl.ds(cid * N + sid * ROWS_PER_IO, ROWS_PER_IO)],
    )

    @pl.when(sid == NS - 1)
    def _copy_tail():
        pltpu.sync_copy(
            acc_sh.at[pl.ds(TAIL_BASE, TAIL_ROWS)],
            out_hbm.at[pl.ds(cid * N + TAIL_BASE, TAIL_ROWS)],
        )


def _sc_agg(h, src, dst):
    mesh = plsc.VectorSubcoreMesh(core_axis_name="c", subcore_axis_name="s")
    return pl.kernel(
        _sc_agg_body,
        out_type=jax.ShapeDtypeStruct((2 * N, D), jnp.float32),
        mesh=mesh,
        scratch_types=[
            pltpu.VMEM((EPW,), jnp.int32),
            pltpu.VMEM((ITERS, K), jnp.int32),
            pltpu.VMEM((2, K, D), jnp.float32),
            pltpu.VMEM_SHARED((N, D), jnp.float32),
            pltpu.SemaphoreType.DMA,
            pltpu.SemaphoreType.DMA,
            pltpu.SemaphoreType.DMA,
        ],
    )(h, src, dst)


BM = 2000


def _split_body(ei_ref, src_ref, dst_ref):
    src_ref[...] = ei_ref[0]
    dst_ref[...] = ei_ref[1]


def _tc_split(ei):
    return pl.pallas_call(
        _split_body,
        out_shape=[jax.ShapeDtypeStruct((E,), jnp.int32),
                   jax.ShapeDtypeStruct((E,), jnp.int32)],
    )(ei)


def _enc_body(x_ref, w_ref, b_ref, o_ref):
    o_ref[...] = jnp.maximum(
        jnp.dot(x_ref[...], w_ref[...], preferred_element_type=jnp.float32)
        + b_ref[...], 0.0)


def _tc_encoder(x, W_enc, b_enc):
    return pl.pallas_call(
        _enc_body,
        grid=(N // BM,),
        in_specs=[
            pl.BlockSpec((BM, D), lambda i: (i, 0)),
            pl.BlockSpec((D, D), lambda i: (0, 0)),
            pl.BlockSpec((1, D), lambda i: (0, 0)),
        ],
        out_specs=pl.BlockSpec((BM, D), lambda i: (i, 0)),
        out_shape=jax.ShapeDtypeStruct((N, D), jnp.float32),
    )(x, W_enc, b_enc.reshape(1, D))


def _tc_mlp(h, agg2, W1, b1, W2, b2, eps, l):
    def body(eps_ref, h_ref, a0_ref, a1_ref, w1_ref, b1_ref, w2_ref,
             b2_ref, o_ref):
        scale = 1.0 + eps_ref[l]
        z = scale * h_ref[...] + a0_ref[...] + a1_ref[...]
        z = jnp.maximum(
            jnp.dot(z, w1_ref[0], preferred_element_type=jnp.float32)
            + b1_ref[l], 0.0)
        o_ref[...] = jnp.maximum(
            jnp.dot(z, w2_ref[0], preferred_element_type=jnp.float32)
            + b2_ref[l], 0.0)

    return pl.pallas_call(
        body,
        grid=(N // BM,),
        in_specs=[
            pl.BlockSpec(memory_space=pltpu.SMEM),
            pl.BlockSpec((BM, D), lambda i: (i, 0)),
            pl.BlockSpec((BM, D), lambda i: (i, 0)),
            pl.BlockSpec((BM, D), lambda i: (i + N // BM, 0)),
            pl.BlockSpec((1, D, D), lambda i: (l, 0, 0)),
            pl.BlockSpec((NLAYERS, D), lambda i: (0, 0)),
            pl.BlockSpec((1, D, D), lambda i: (l, 0, 0)),
            pl.BlockSpec((NLAYERS, D), lambda i: (0, 0)),
        ],
        out_specs=pl.BlockSpec((BM, D), lambda i: (i, 0)),
        out_shape=jax.ShapeDtypeStruct((N, D), jnp.float32),
    )(eps, h, agg2, agg2, W1, b1, W2, b2)


def kernel(x, edge_index, W_enc, b_enc, W1, b1, W2, b2, eps):
    src, dst = _tc_split(edge_index.astype(jnp.int32))
    h = _tc_encoder(x, W_enc, b_enc)
    for l in range(NLAYERS):
        agg2 = _sc_agg(h, src, dst)
        h = _tc_mlp(h, agg2, W1, b1, W2, b2, eps, l)
    return h
